# baseline (speedup 1.0000x reference)
import functools

import jax
import jax.numpy as jnp
from jax import lax
from jax.experimental import pallas as pl
from jax.experimental.pallas import tpu as pltpu
from jax.experimental.pallas import tpu_sc as plsc

CHUNK = 128
DEPTH = 2
ALPHA = 0.3


def _pack_words(x):
    b2, uu = x.shape
    bits = lax.bitcast_convert_type(x, jnp.uint32)
    r = (bits + jnp.uint32(0x8000)) >> 16
    r4 = r.reshape(b2, uu // 32, 2, 16)
    w = r4[:, :, 0, :] | (r4[:, :, 1, :] << 16)
    return lax.bitcast_convert_type(w.reshape(b2, uu // 2), jnp.int32)


def _dense_body(x_ref, w_ref, b_ref, h_ref, u_ref, up_ref):
    h = jnp.dot(x_ref[...], w_ref[...], preferred_element_type=jnp.float32)
    h = h + b_ref[...]
    h_ref[...] = h
    un = jnp.exp(jnp.where(h > 0, h, ALPHA * h))
    u_ref[...] = un
    up_ref[...] = _pack_words(un)


def _attg_body(u_ref, n_ref, h_ref, att_ref, g_ref):
    att = u_ref[...] / n_ref[...]
    att_ref[...] = _pack_words(att)
    g_ref[...] = _pack_words(h_ref[...] * att)


def _final_body(num_ref, den_ref, o_ref):
    o_ref[...] = jnp.maximum(num_ref[...] / den_ref[...], 0.0)


def _tc_dense(x, w, b, blk):
    n, f = x.shape
    u = w.shape[1]
    grid = n // blk
    return pl.pallas_call(
        _dense_body,
        grid=(grid,),
        in_specs=[
            pl.BlockSpec((blk, f), lambda i: (i, 0)),
            pl.BlockSpec((f, u), lambda i: (0, 0)),
            pl.BlockSpec((1, u), lambda i: (0, 0)),
        ],
        out_specs=[
            pl.BlockSpec((blk, u), lambda i: (i, 0)),
            pl.BlockSpec((blk, u), lambda i: (i, 0)),
            pl.BlockSpec((blk, u // 2), lambda i: (i, 0)),
        ],
        out_shape=[
            jax.ShapeDtypeStruct((n, u), jnp.float32),
            jax.ShapeDtypeStruct((n, u), jnp.float32),
            jax.ShapeDtypeStruct((n, u // 2), jnp.int32),
        ],
    )(x, w, b.reshape(1, u))


def _tc_elementwise3(body, a, b_, c, outs, blk):
    n, u = a.shape
    grid = n // blk
    spec = pl.BlockSpec((blk, u), lambda i: (i, 0))
    out_shape = [jax.ShapeDtypeStruct((n, wd), d) for d, wd in outs]
    out_specs = [pl.BlockSpec((blk, wd), lambda i: (i, 0))
                 for _, wd in outs]
    if len(outs) == 1:
        out_shape = out_shape[0]
        out_specs = out_specs[0]
    args = [x for x in (a, b_, c) if x is not None]
    return pl.pallas_call(
        body,
        grid=(grid,),
        in_specs=[spec] * len(args),
        out_specs=out_specs,
        out_shape=out_shape,
    )(*args)


RPT = 312
TRASH = RPT + 16
ACC_ROWS = RPT + 24


def _sc_segsum(table, colp, rlocp, meta, n_nodes, n_units, e_edges):
    mesh = plsc.VectorSubcoreMesh(core_axis_name="c", subcore_axis_name="s")
    jg = n_units // 16

    @functools.partial(
        pl.kernel,
        out_type=jax.ShapeDtypeStruct((n_nodes, n_units), jnp.float32),
        mesh=mesh,
        compiler_params=pltpu.CompilerParams(needs_layout_passes=False),
        scratch_types=[
            [pltpu.VMEM((CHUNK,), jnp.int32)] * DEPTH,
            [pltpu.VMEM((CHUNK,), jnp.int32)] * DEPTH,
            pltpu.VMEM((16,), jnp.int32),
            [pltpu.VMEM((CHUNK, n_units // 2), jnp.int32)] * DEPTH,
            pltpu.VMEM((ACC_ROWS, n_units), jnp.float32),
            [pltpu.SemaphoreType.DMA] * DEPTH,
            [pltpu.SemaphoreType.DMA] * DEPTH,
        ],
    )
    def k(table_h, col_h, rloc_h, meta_h, out_h,
          cidx, ridx, mvec, rows, acc, isem, gsem):
        c = lax.axis_index("c")
        s = lax.axis_index("s")
        w = c * 16 + s

        def _zrow(i, _):
            for j in range(jg):
                acc[i, pl.ds(j * 16, 16)] = jnp.zeros((16,), jnp.float32)
            return 0
        lax.fori_loop(0, ACC_ROWS, _zrow, 0)

        pltpu.sync_copy(meta_h.at[w], mvec)
        mv = mvec[...]
        start_a = pl.multiple_of(mv[0], 8)
        start = mv[1]
        end = mv[2]
        nch = (jnp.maximum(end - start_a, 0) + CHUNK - 1) // CHUNK

        def _load_idx(i, p):
            base = start_a + i * CHUNK
            pltpu.async_copy(col_h.at[pl.ds(base, CHUNK)], cidx[p], isem[p])
            pltpu.async_copy(rloc_h.at[pl.ds(base, CHUNK)], ridx[p], isem[p])

        def _wait_idx(p):
            pltpu.make_async_copy(col_h.at[pl.ds(0, CHUNK)],
                                  cidx[p], isem[p]).wait()
            pltpu.make_async_copy(rloc_h.at[pl.ds(0, CHUNK)],
                                  ridx[p], isem[p]).wait()

        def _gather(p):
            pltpu.async_copy(table_h.at[cidx[p]], rows[p], gsem[p])

        def _wait_gather(p):
            pltpu.make_async_copy(table_h.at[cidx[p]],
                                  rows[p], gsem[p]).wait()

        def _accum(i, p):
            base = start_a + i * CHUNK

            @plsc.parallel_loop(0, CHUNK, step=16)
            def _grp(e0):
                lv = ridx[p][pl.ds(e0, 16)]
                lks = []
                for kk in range(16):
                    pos = base + e0 + kk
                    ok = (pos >= start) & (pos < end)
                    lks.append(jnp.where(ok, lv[kk], TRASH))
                for kk in range(16):
                    er = e0 + kk
                    vals = []
                    for m2 in range(jg // 2):
                        wd = rows[p][er, pl.ds(m2 * 16, 16)]
                        vals.append(plsc.bitcast(wd << 16, jnp.float32))
                        vals.append(plsc.bitcast(
                            wd & jnp.int32(-65536), jnp.float32))
                    for m in range(jg):
                        plsc.addupdate(acc.at[lks[kk], pl.ds(m * 16, 16)],
                                       vals[m])

        for p in range(DEPTH):
            _load_idx(jnp.int32(p), p)
        for p in range(DEPTH - 1):
            _wait_idx(p)
            _gather(p)
        nd = (nch + DEPTH - 1) // DEPTH

        def _iter(j, _):
            for q in range(DEPTH):
                i = j * DEPTH + q
                pg = (q + DEPTH - 1) % DEPTH
                _wait_idx(pg)
                _gather(pg)
                _wait_gather(q)
                _accum(i, q)
                _load_idx(i + DEPTH, q)
            return 0

        lax.fori_loop(0, nd, _iter, 0)
        for p in range(DEPTH - 1):
            _wait_gather(p)
        _wait_idx(DEPTH - 1)

        pltpu.sync_copy(acc.at[pl.ds(0, RPT)],
                        out_h.at[pl.ds(w * RPT, RPT)])

        @pl.when(w == 31)
        def _():
            pltpu.sync_copy(acc.at[pl.ds(RPT, 16)],
                            out_h.at[pl.ds(32 * RPT, 16)])

    return k(table, colp, rlocp, meta)


def kernel(node_ids, node_features, W, b):
    n, f = node_features.shape
    u = W.shape[1]
    e = node_ids.shape[1]
    half = n // 2

    row = node_ids[0]
    col = node_ids[1]

    hidden, unnorm, unnorm_p = _tc_dense(node_features, W, b, blk=400)

    e_pad = e + 8 * CHUNK
    bvals = jnp.concatenate([jnp.arange(32, dtype=jnp.int32) * RPT,
                             jnp.array([n], jnp.int32)])
    bnd = jnp.searchsorted(row, bvals, side="left").astype(jnp.int32)
    meta = jnp.stack([bnd[:32] // 8 * 8, bnd[:32], bnd[1:]], axis=1)
    meta = jnp.pad(meta, ((0, 0), (0, 13)))
    tile_of_row = jnp.minimum(row // RPT, 31)
    rloc = row - tile_of_row * RPT
    pad = e_pad - e
    colp = jnp.pad(col, (0, pad))
    rlocp = jnp.pad(rloc, (0, pad), constant_values=TRASH)

    norm = _sc_segsum(unnorm_p, colp, rlocp, meta, n, u, e)
    att_p, g_p = _tc_elementwise3(_attg_body, unnorm, norm, hidden,
                                  [(jnp.int32, u // 2), (jnp.int32, u // 2)],
                                  blk=400)
    den = _sc_segsum(att_p, colp, rlocp, meta, n, u, e)
    num = _sc_segsum(g_p, colp, rlocp, meta, n, u, e)
    out = _tc_elementwise3(_final_body, num, den, None,
                           [(jnp.float32, u)], blk=400)
    return out
